# baseline (speedup 1.0000x reference)
import functools

import jax
import jax.numpy as jnp
from jax import lax
from jax.experimental import pallas as pl
from jax.experimental.pallas import tpu as pltpu
from jax.experimental.pallas import tpu_sc as plsc

_HIGH = lax.Precision.HIGHEST


def _score_body(tb_ref, w_ref, b_ref, out_ref):
    s = jnp.dot(tb_ref[...], w_ref[...],
                preferred_element_type=jnp.float32) + b_ref[...]
    blk, c = s.shape
    cp = out_ref.shape[1]
    out_ref[...] = jnp.concatenate(
        [s, jnp.zeros((blk, cp - c), jnp.float32)], axis=1)


def _final_body(s_ref, hv_ref, tv_ref, rc_ref, lab_ref, lwt_ref,
                lb_ref, logits_ref, loss_ref, *, IB, L, D, R, H, C):
    i = pl.program_id(0)
    ng = pl.num_programs(0)
    rc_m = rc_ref[...].reshape(IB, 1)
    iota_c = lax.broadcasted_iota(jnp.int32, (1, C), 1)
    iota_r = lax.broadcasted_iota(jnp.int32, (1, R), 1)
    wsel = jnp.where(iota_c // H == rc_m, 1.0 / H, 0.0)
    onehot = jnp.where(iota_r == rc_m, 1.0, 0.0)

    def side(s, v):
        x = jnp.maximum(s, 0.0)
        m = jnp.max(x, axis=0, keepdims=True)
        e = jnp.exp(x - m)
        q = e / jnp.sum(e, axis=0, keepdims=True)
        qbar = jnp.sum(q * wsel[None, :, :], axis=2)
        return jnp.sum(qbar[:, :, None] * v, axis=0)

    oh = side(s_ref[0, :, :, 0:C], hv_ref[...])
    ot = side(s_ref[1, :, :, 0:C], tv_ref[...])
    lwt = lwt_ref[...]
    la = (jnp.dot(oh, lwt[:D], preferred_element_type=jnp.float32,
                  precision=_HIGH)
          + jnp.dot(ot, lwt[D:], preferred_element_type=jnp.float32,
                    precision=_HIGH))
    l0 = jnp.sum(la[:, 0:R] * onehot, axis=1, keepdims=True)
    l1 = jnp.sum(la[:, R:2 * R] * onehot, axis=1, keepdims=True)
    logits = jnp.concatenate([l0, l1], axis=1) + jnp.dot(
        onehot, lb_ref[...], preferred_element_type=jnp.float32,
        precision=_HIGH)
    logits_ref[...] = logits

    m2 = jnp.max(logits, axis=1, keepdims=True)
    lse = m2 + jnp.log(jnp.sum(jnp.exp(logits - m2), axis=1, keepdims=True))
    lab_m = lab_ref[...].reshape(IB, 1)
    laboh = jnp.where(lax.broadcasted_iota(jnp.int32, (1, 2), 1) == lab_m,
                      1.0, 0.0)
    sel = jnp.sum(logits * laboh, axis=1, keepdims=True)
    part = jnp.sum(lse - sel)

    @pl.when(i == 0)
    def _():
        loss_ref[...] = jnp.zeros_like(loss_ref)

    loss_ref[...] += part

    @pl.when(i == ng - 1)
    def _():
        loss_ref[...] *= 1.0 / (IB * ng)


def _make_sc_gather(V, C, CP, NW, NCH, CH):
    mesh = plsc.VectorSubcoreMesh(core_axis_name="c", subcore_axis_name="s")

    @functools.partial(
        pl.kernel,
        mesh=mesh,
        out_type=jax.ShapeDtypeStruct((NW, NCH, CH, CP), jnp.float32),
        scratch_types=[
            pltpu.VMEM((NCH, CH), jnp.int32),
            pltpu.VMEM((2, CH, CP), jnp.float32),
            pltpu.SemaphoreType.DMA,
            pltpu.SemaphoreType.DMA,
        ],
    )
    def gather(scores_hbm, ids_hbm, out_hbm, idx_v, rows_v, gsem, ssem):
        wid = lax.axis_index("s") * 2 + lax.axis_index("c")
        pltpu.sync_copy(ids_hbm.at[wid], idx_v)

        def body(i, carry):
            k0 = 2 * i

            @pl.when(i > 0)
            def _():
                for b in range(2):
                    pltpu.make_async_copy(
                        rows_v.at[b], out_hbm.at[wid, k0 + b], ssem).wait()

            for b in range(2):
                pltpu.make_async_copy(
                    scores_hbm.at[idx_v.at[k0 + b]], rows_v.at[b],
                    gsem).start()
            for b in range(2):
                pltpu.make_async_copy(
                    scores_hbm.at[idx_v.at[k0 + b]], rows_v.at[b],
                    gsem).wait()
                pltpu.make_async_copy(
                    rows_v.at[b], out_hbm.at[wid, k0 + b], ssem).start()
            return carry

        lax.fori_loop(0, NCH // 2, body, 0)
        for b in range(2):
            pltpu.make_async_copy(
                rows_v.at[b], out_hbm.at[wid, NCH - 2 + b], ssem).wait()

    return gather


def kernel(head_ids, tail_ids, head_vals, tail_vals, nd_labels, rc_ids,
           table, gat_w, gat_b, lin_w, lin_b):
    B, L = head_ids.shape
    V, D = table.shape
    H, R = gat_w.shape[0], gat_w.shape[1]
    C = H * R

    CP = 128

    w_all = gat_w[:, :, 0, :].transpose(2, 1, 0).reshape(D, C)
    b_all = gat_b[:, :, 0].T.reshape(1, C)
    lwt = lin_w.transpose(2, 1, 0).reshape(2 * D, 2 * R)

    BLK_V = 2000
    assert V % BLK_V == 0
    scores = pl.pallas_call(
        _score_body,
        grid=(V // BLK_V,),
        in_specs=[
            pl.BlockSpec((BLK_V, D), lambda i: (i, 0)),
            pl.BlockSpec((D, C), lambda i: (0, 0)),
            pl.BlockSpec((1, C), lambda i: (0, 0)),
        ],
        out_specs=pl.BlockSpec((BLK_V, CP), lambda i: (i, 0)),
        out_shape=jax.ShapeDtypeStruct((V, CP), jnp.float32),
    )(table, w_all, b_all)

    info = plsc.get_sparse_core_info()
    NW = info.num_cores * info.num_subcores
    total = 2 * B * L
    CH = 80
    NCH = total // (NW * CH)
    assert NW * NCH * CH == total
    ids_lm = jnp.concatenate(
        [head_ids.T.reshape(-1), tail_ids.T.reshape(-1)]).astype(jnp.int32)
    ids3 = ids_lm.reshape(NW, NCH, CH)
    gathered = _make_sc_gather(V, C, CP, NW, NCH, CH)(scores, ids3)
    s_all = gathered.reshape(2, L, B, CP)

    IB = 64
    G = B // IB
    rc2 = rc_ids.astype(jnp.int32).reshape(G, 1, IB)
    lab2 = nd_labels.astype(jnp.int32).reshape(G, 1, IB)
    logits, loss_sum = pl.pallas_call(
        functools.partial(_final_body, IB=IB, L=L, D=D, R=R, H=H, C=C),
        grid=(G,),
        in_specs=[
            pl.BlockSpec((2, L, IB, CP), lambda i: (0, 0, i, 0)),
            pl.BlockSpec((L, IB, D), lambda i: (0, i, 0)),
            pl.BlockSpec((L, IB, D), lambda i: (0, i, 0)),
            pl.BlockSpec((1, 1, IB), lambda i: (i, 0, 0)),
            pl.BlockSpec((1, 1, IB), lambda i: (i, 0, 0)),
            pl.BlockSpec((2 * D, 2 * R), lambda i: (0, 0)),
            pl.BlockSpec((R, 2), lambda i: (0, 0)),
        ],
        out_specs=[
            pl.BlockSpec((IB, 2), lambda i: (i, 0)),
            pl.BlockSpec((1, 1), lambda i: (0, 0)),
        ],
        out_shape=[
            jax.ShapeDtypeStruct((B, 2), jnp.float32),
            jax.ShapeDtypeStruct((1, 1), jnp.float32),
        ],
    )(s_all, head_vals.transpose(1, 0, 2), tail_vals.transpose(1, 0, 2),
      rc2, lab2, lwt, lin_b)

    return (logits, loss_sum.reshape(()))


# speedup vs baseline: 1.0064x; 1.0064x over previous
import functools

import jax
import jax.numpy as jnp
from jax import lax
from jax.experimental import pallas as pl
from jax.experimental.pallas import tpu as pltpu
from jax.experimental.pallas import tpu_sc as plsc

_HIGH = lax.Precision.HIGHEST


def _score_body(tb_ref, w_ref, b_ref, out_ref):
    s = jnp.dot(tb_ref[...], w_ref[...],
                preferred_element_type=jnp.float32) + b_ref[...]
    blk, c = s.shape
    cp = out_ref.shape[1]
    out_ref[...] = jnp.concatenate(
        [s, jnp.zeros((blk, cp - c), jnp.float32)], axis=1)


def _final_body(s_ref, hv_ref, tv_ref, rc_ref, lab_ref, lwt_ref,
                lb_ref, logits_ref, loss_ref, *, IB, L, D, R, H, C):
    i = pl.program_id(0)
    ng = pl.num_programs(0)
    rc_m = rc_ref[...].reshape(IB, 1)
    iota_c = lax.broadcasted_iota(jnp.int32, (1, C), 1)
    iota_r = lax.broadcasted_iota(jnp.int32, (1, R), 1)
    wsel = jnp.where(iota_c // H == rc_m, 1.0 / H, 0.0)
    onehot = jnp.where(iota_r == rc_m, 1.0, 0.0)

    def side(s, v):
        x = jnp.maximum(s, 0.0)
        m = jnp.max(x, axis=0, keepdims=True)
        e = jnp.exp(x - m)
        q = e / jnp.sum(e, axis=0, keepdims=True)
        qbar = jnp.sum(q * wsel[None, :, :], axis=2)
        return jnp.sum(qbar[:, :, None] * v, axis=0)

    oh = side(s_ref[0, :, :, 0:C], hv_ref[...])
    ot = side(s_ref[1, :, :, 0:C], tv_ref[...])
    lwt = lwt_ref[...]
    la = (jnp.dot(oh, lwt[:D], preferred_element_type=jnp.float32,
                  precision=_HIGH)
          + jnp.dot(ot, lwt[D:], preferred_element_type=jnp.float32,
                    precision=_HIGH))
    l0 = jnp.sum(la[:, 0:R] * onehot, axis=1, keepdims=True)
    l1 = jnp.sum(la[:, R:2 * R] * onehot, axis=1, keepdims=True)
    logits = jnp.concatenate([l0, l1], axis=1) + jnp.dot(
        onehot, lb_ref[...], preferred_element_type=jnp.float32,
        precision=_HIGH)
    logits_ref[...] = logits

    m2 = jnp.max(logits, axis=1, keepdims=True)
    lse = m2 + jnp.log(jnp.sum(jnp.exp(logits - m2), axis=1, keepdims=True))
    lab_m = lab_ref[...].reshape(IB, 1)
    laboh = jnp.where(lax.broadcasted_iota(jnp.int32, (1, 2), 1) == lab_m,
                      1.0, 0.0)
    sel = jnp.sum(logits * laboh, axis=1, keepdims=True)
    part = jnp.sum(lse - sel)

    @pl.when(i == 0)
    def _():
        loss_ref[...] = jnp.zeros_like(loss_ref)

    loss_ref[...] += part

    @pl.when(i == ng - 1)
    def _():
        loss_ref[...] *= 1.0 / (IB * ng)


def _make_sc_gather(V, C, CP, NW, NCH, CH):
    mesh = plsc.VectorSubcoreMesh(core_axis_name="c", subcore_axis_name="s")

    @functools.partial(
        pl.kernel,
        mesh=mesh,
        out_type=jax.ShapeDtypeStruct((NW, NCH, CH, CP), jnp.float32),
        scratch_types=[
            pltpu.VMEM((NCH, CH), jnp.int32),
            pltpu.VMEM((2, CH, CP), jnp.float32),
            pltpu.SemaphoreType.DMA,
            pltpu.SemaphoreType.DMA,
        ],
    )
    def gather(scores_hbm, ids_hbm, out_hbm, idx_v, rows_v, gsem, ssem):
        wid = lax.axis_index("s") * 2 + lax.axis_index("c")
        pltpu.sync_copy(ids_hbm.at[wid], idx_v)

        def body(i, carry):
            k0 = 2 * i

            @pl.when(i > 0)
            def _():
                for b in range(2):
                    pltpu.make_async_copy(
                        rows_v.at[b], out_hbm.at[wid, k0 + b], ssem).wait()

            for b in range(2):
                pltpu.make_async_copy(
                    scores_hbm.at[idx_v.at[k0 + b]], rows_v.at[b],
                    gsem).start()
            for b in range(2):
                pltpu.make_async_copy(
                    scores_hbm.at[idx_v.at[k0 + b]], rows_v.at[b],
                    gsem).wait()
                pltpu.make_async_copy(
                    rows_v.at[b], out_hbm.at[wid, k0 + b], ssem).start()
            return carry

        lax.fori_loop(0, NCH // 2, body, 0)
        for b in range(2):
            pltpu.make_async_copy(
                rows_v.at[b], out_hbm.at[wid, NCH - 2 + b], ssem).wait()

    return gather


def kernel(head_ids, tail_ids, head_vals, tail_vals, nd_labels, rc_ids,
           table, gat_w, gat_b, lin_w, lin_b):
    B, L = head_ids.shape
    V, D = table.shape
    H, R = gat_w.shape[0], gat_w.shape[1]
    C = H * R

    CP = 128

    w_all = gat_w[:, :, 0, :].transpose(2, 1, 0).reshape(D, C)
    b_all = gat_b[:, :, 0].T.reshape(1, C)
    lwt = lin_w.transpose(2, 1, 0).reshape(2 * D, 2 * R)

    BLK_V = 2000
    assert V % BLK_V == 0
    scores = pl.pallas_call(
        _score_body,
        grid=(V // BLK_V,),
        in_specs=[
            pl.BlockSpec((BLK_V, D), lambda i: (i, 0)),
            pl.BlockSpec((D, C), lambda i: (0, 0)),
            pl.BlockSpec((1, C), lambda i: (0, 0)),
        ],
        out_specs=pl.BlockSpec((BLK_V, CP), lambda i: (i, 0)),
        out_shape=jax.ShapeDtypeStruct((V, CP), jnp.float32),
    )(table, w_all, b_all)

    info = plsc.get_sparse_core_info()
    NW = info.num_cores * info.num_subcores
    total = 2 * B * L
    CH = 80
    NCH = total // (NW * CH)
    assert NW * NCH * CH == total
    ids_lm = jnp.concatenate(
        [head_ids.T.reshape(-1), tail_ids.T.reshape(-1)]).astype(jnp.int32)
    ids3 = ids_lm.reshape(NW, NCH, CH)
    gathered = _make_sc_gather(V, C, CP, NW, NCH, CH)(scores, ids3)
    s_all = gathered.reshape(2, L, B, CP)

    IB = 32
    G = B // IB
    rc2 = rc_ids.astype(jnp.int32).reshape(G, 1, IB)
    lab2 = nd_labels.astype(jnp.int32).reshape(G, 1, IB)
    logits, loss_sum = pl.pallas_call(
        functools.partial(_final_body, IB=IB, L=L, D=D, R=R, H=H, C=C),
        grid=(G,),
        in_specs=[
            pl.BlockSpec((2, L, IB, CP), lambda i: (0, 0, i, 0)),
            pl.BlockSpec((L, IB, D), lambda i: (0, i, 0)),
            pl.BlockSpec((L, IB, D), lambda i: (0, i, 0)),
            pl.BlockSpec((1, 1, IB), lambda i: (i, 0, 0)),
            pl.BlockSpec((1, 1, IB), lambda i: (i, 0, 0)),
            pl.BlockSpec((2 * D, 2 * R), lambda i: (0, 0)),
            pl.BlockSpec((R, 2), lambda i: (0, 0)),
        ],
        out_specs=[
            pl.BlockSpec((IB, 2), lambda i: (i, 0)),
            pl.BlockSpec((1, 1), lambda i: (0, 0)),
        ],
        out_shape=[
            jax.ShapeDtypeStruct((B, 2), jnp.float32),
            jax.ShapeDtypeStruct((1, 1), jnp.float32),
        ],
    )(s_all, head_vals.transpose(1, 0, 2), tail_vals.transpose(1, 0, 2),
      rc2, lab2, lwt, lin_b)

    return (logits, loss_sum.reshape(()))
